# baseline (speedup 1.0000x reference)
import functools

import jax
import jax.numpy as jnp
from jax import lax
from jax.experimental import pallas as pl
from jax.experimental.pallas import tpu as pltpu
from jax.experimental.pallas import tpu_sc as plsc


def _routing(topk_ids, topk_weights, E, BLK_M):
    T = topk_ids.shape[0]
    NB = T // BLK_M
    P = NB + E
    eid = topk_ids[:, 0].astype(jnp.int32)
    w = topk_weights[:, 0].astype(jnp.float32)
    C = 16
    L = T // C
    oh = (eid[:, None] == jnp.arange(E, dtype=jnp.int32)[None, :]).astype(
        jnp.float32
    )
    oh3 = oh.reshape(C, L, E)
    tril = jnp.tril(jnp.ones((L, L), jnp.float32))
    local = jnp.matmul(tril, oh3)
    chunk_tot = oh3.sum(axis=1)
    offs = jnp.cumsum(chunk_tot, axis=0) - chunk_tot
    cum = local + offs[:, None, :]
    counts = chunk_tot.sum(axis=0).astype(jnp.int32)
    ranks = (cum * oh3).sum(axis=-1).reshape(T).astype(jnp.int32) - 1
    tok_end = jnp.cumsum(counts)
    tok_start = tok_end - counts
    base = (oh * tok_start[None, :].astype(jnp.float32)).sum(-1).astype(
        jnp.int32
    )
    pos = base + ranks
    w_sorted = jnp.zeros((T,), jnp.float32).at[pos].set(w)

    rs = jnp.arange(NB, dtype=jnp.int32) * BLK_M
    ovl = (jnp.minimum(tok_end[None, :], (rs + BLK_M)[:, None])
           - jnp.maximum(tok_start[None, :], rs[:, None]))
    flat = (ovl > 0).reshape(-1)
    dest = jnp.cumsum(flat.astype(jnp.int32)) - 1
    npairs = flat.astype(jnp.int32).sum()
    pr = jnp.arange(P, dtype=jnp.int32)
    pair_oh = ((dest[:, None] == pr[None, :]) & flat[:, None]).astype(
        jnp.int32
    )
    fi = jnp.arange(NB * E, dtype=jnp.int32)
    gb = (pair_oh * (fi // E)[:, None]).sum(axis=0)
    ge = (pair_oh * (fi % E)[:, None]).sum(axis=0)
    ge_last = jnp.max(jnp.where(flat, fi % E, -1))
    unused = pr >= npairs
    gb = jnp.where(unused, NB - 1, gb).astype(jnp.int32)
    ge = jnp.where(unused, ge_last, ge).astype(jnp.int32)
    first = jnp.concatenate(
        [jnp.ones((1,), jnp.int32), (gb[1:] != gb[:-1]).astype(jnp.int32)]
    )
    rowidx = gb[:, None] * BLK_M + jnp.arange(BLK_M, dtype=jnp.int32)[None, :]
    seg = (rowidx >= tok_start[ge][:, None]) & (rowidx < tok_end[ge][:, None])
    pw = jnp.where(seg & ~unused[:, None],
                   jnp.take(w_sorted.reshape(NB, BLK_M), gb, axis=0),
                   0.0)
    return pos, gb, ge, first, pw


def _mm_body(gb_ref, ge_ref, first_ref, xs_ref, *rest):
    ws, pw_ref, ys_ref = rest[:-2], rest[-2], rest[-1]
    p = pl.program_id(0)
    xb = xs_ref[...]
    acc = None
    for j, wj in enumerate(ws):
        kj = wj.shape[-2]
        part = jnp.dot(xb[:, j * kj:(j + 1) * kj], wj[0, 0],
                       preferred_element_type=jnp.float32)
        acc = part if acc is None else acc + part
    contrib = acc * pw_ref[0]

    @pl.when(first_ref[p] == 1)
    def _init():
        ys_ref[...] = contrib

    @pl.when(first_ref[p] == 0)
    def _accum():
        ys_ref[...] += contrib


def kernel(x, topk_weights, topk_ids, W1):
    T, K = x.shape
    E, _, N = W1.shape
    BLK_M = 128
    NB = T // BLK_M
    P = NB + E

    pos, gb, ge, first, pw = _routing(topk_ids, topk_weights, E, BLK_M)

    info = plsc.get_sparse_core_info()
    NC, NS = info.num_cores, info.num_subcores
    NW = NC * NS
    CH = T // NW
    mesh = plsc.VectorSubcoreMesh(core_axis_name="c", subcore_axis_name="s")

    @functools.partial(
        pl.kernel,
        out_type=jax.ShapeDtypeStruct((T, K), jnp.float32),
        mesh=mesh,
        scratch_types=[
            pltpu.VMEM((CH,), jnp.int32),
            pltpu.VMEM((CH, K), jnp.float32),
            pltpu.SemaphoreType.DMA,
        ],
    )
    def dispatch(x_hbm, pos_hbm, xs_hbm, idx_v, rows_v, sem):
        wid = lax.axis_index("s") * NC + lax.axis_index("c")
        base = wid * CH
        pltpu.sync_copy(pos_hbm.at[pl.ds(base, CH)], idx_v)
        pltpu.sync_copy(x_hbm.at[pl.ds(base, CH)], rows_v)
        pltpu.async_copy(rows_v, xs_hbm.at[idx_v], sem).wait()

    @functools.partial(
        pl.kernel,
        out_type=jax.ShapeDtypeStruct((T, N), jnp.float32),
        mesh=mesh,
        scratch_types=[
            pltpu.VMEM((CH,), jnp.int32),
            pltpu.VMEM((CH, N), jnp.float32),
            pltpu.SemaphoreType.DMA,
        ],
    )
    def combine(ys_hbm, pos_hbm, out_hbm, idx_v, rows_v, sem):
        wid = lax.axis_index("s") * NC + lax.axis_index("c")
        base = wid * CH
        pltpu.sync_copy(pos_hbm.at[pl.ds(base, CH)], idx_v)
        pltpu.async_copy(ys_hbm.at[idx_v], rows_v, sem).wait()
        pltpu.sync_copy(rows_v, out_hbm.at[pl.ds(base, CH)])

    xs = dispatch(x, pos)

    S = 1
    KS_ = K // S
    W1r = W1.reshape(E, S, KS_, N)

    def _w_spec(j):
        return pl.BlockSpec(
            (1, 1, KS_, N), lambda p, gb, ge, fr, j=j: (ge[p], j, 0, 0)
        )

    grid_spec = pltpu.PrefetchScalarGridSpec(
        num_scalar_prefetch=3,
        grid=(P,),
        in_specs=[
            pl.BlockSpec((BLK_M, K), lambda p, gb, ge, fr: (gb[p], 0)),
            *[_w_spec(j) for j in range(S)],
            pl.BlockSpec((1, BLK_M, 1), lambda p, gb, ge, fr: (p, 0, 0)),
        ],
        out_specs=pl.BlockSpec((BLK_M, N), lambda p, gb, ge, fr: (gb[p], 0)),
    )
    ys = pl.pallas_call(
        _mm_body,
        grid_spec=grid_spec,
        out_shape=jax.ShapeDtypeStruct((T, N), jnp.float32),
        compiler_params=pltpu.CompilerParams(
            dimension_semantics=("arbitrary",),
        ),
    )(gb, ge, first, xs, *([W1r] * S), pw.reshape(P, BLK_M, 1))

    return combine(ys, pos)


# speedup vs baseline: 1.0533x; 1.0533x over previous
import functools

import jax
import jax.numpy as jnp
from jax import lax
from jax.experimental import pallas as pl
from jax.experimental.pallas import tpu as pltpu
from jax.experimental.pallas import tpu_sc as plsc


def _routing(topk_ids, topk_weights, E, BLK_M):
    T = topk_ids.shape[0]
    NB = T // BLK_M
    P = NB + E
    eid = topk_ids[:, 0].astype(jnp.int32)
    w = topk_weights[:, 0].astype(jnp.float32)
    C = 16
    L = T // C
    oh = (eid[:, None] == jnp.arange(E, dtype=jnp.int32)[None, :]).astype(
        jnp.float32
    )
    oh3 = oh.reshape(C, L, E)
    tril = jnp.tril(jnp.ones((L, L), jnp.float32))
    local = jnp.matmul(tril, oh3)
    chunk_tot = oh3.sum(axis=1)
    offs = jnp.cumsum(chunk_tot, axis=0) - chunk_tot
    cum = local + offs[:, None, :]
    counts = chunk_tot.sum(axis=0).astype(jnp.int32)
    ranks = (cum * oh3).sum(axis=-1).reshape(T).astype(jnp.int32) - 1
    tok_end = jnp.cumsum(counts)
    tok_start = tok_end - counts
    base = (oh * tok_start[None, :].astype(jnp.float32)).sum(-1).astype(
        jnp.int32
    )
    pos = base + ranks

    rs = jnp.arange(NB, dtype=jnp.int32) * BLK_M
    ovl = (jnp.minimum(tok_end[None, :], (rs + BLK_M)[:, None])
           - jnp.maximum(tok_start[None, :], rs[:, None]))
    flat = (ovl > 0).reshape(-1)
    dest = jnp.cumsum(flat.astype(jnp.int32)) - 1
    npairs = flat.astype(jnp.int32).sum()
    pr = jnp.arange(P, dtype=jnp.int32)
    pair_oh = ((dest[:, None] == pr[None, :]) & flat[:, None]).astype(
        jnp.float32
    )
    fi = jnp.arange(NB * E, dtype=jnp.int32)
    bev = jnp.stack([(fi // E).astype(jnp.float32),
                     (fi % E).astype(jnp.float32)], axis=1)
    res = jnp.einsum("fp,fc->pc", pair_oh, bev)
    gb_r = res[:, 0].astype(jnp.int32)
    ge_r = res[:, 1].astype(jnp.int32)
    ge_last = jnp.max(jnp.where(flat, fi % E, -1))
    unused = pr >= npairs
    gb = jnp.where(unused, NB - 1, gb_r).astype(jnp.int32)
    ge = jnp.where(unused, ge_last, ge_r).astype(jnp.int32)
    first = jnp.concatenate(
        [jnp.ones((1,), jnp.int32), (gb[1:] != gb[:-1]).astype(jnp.int32)]
    )
    rowidx = gb[:, None] * BLK_M + jnp.arange(BLK_M, dtype=jnp.int32)[None, :]
    seg = ((rowidx >= tok_start[ge][:, None]) & (rowidx < tok_end[ge][:, None])
           & ~unused[:, None]).astype(jnp.float32)
    return pos, gb, ge, first, seg


def _mm_body(gb_ref, ge_ref, first_ref, xs_ref, *rest):
    ws, seg_ref, ys_ref = rest[:-2], rest[-2], rest[-1]
    p = pl.program_id(0)
    acc = None
    koff = 0
    for wj in ws:
        kj = wj.shape[-2]
        part = jnp.dot(xs_ref[:, koff:koff + kj], wj[0, 0],
                       preferred_element_type=jnp.float32)
        acc = part if acc is None else acc + part
        koff += kj
    wcol = xs_ref[:, koff:koff + 1]
    contrib = acc * (wcol * seg_ref[0])

    @pl.when(first_ref[p] == 1)
    def _init():
        ys_ref[...] = contrib

    @pl.when(first_ref[p] == 0)
    def _accum():
        ys_ref[...] += contrib


def kernel(x, topk_weights, topk_ids, W1):
    T, K = x.shape
    E, _, N = W1.shape
    BLK_M = 128
    NB = T // BLK_M
    P = NB + E

    pos, gb, ge, first, seg = _routing(topk_ids, topk_weights, E, BLK_M)
    wtok = jnp.broadcast_to(
        topk_weights.astype(jnp.float32), (T, 128)
    )

    info = plsc.get_sparse_core_info()
    NC, NS = info.num_cores, info.num_subcores
    NW = NC * NS
    CH = T // NW
    Kp = K + 128
    mesh = plsc.VectorSubcoreMesh(core_axis_name="c", subcore_axis_name="s")

    @functools.partial(
        pl.kernel,
        out_type=jax.ShapeDtypeStruct((T, Kp), jnp.float32),
        mesh=mesh,
        scratch_types=[
            pltpu.VMEM((CH,), jnp.int32),
            pltpu.VMEM((CH, Kp), jnp.float32),
            pltpu.SemaphoreType.DMA,
        ],
    )
    def dispatch(x_hbm, w_hbm, pos_hbm, xs_hbm, idx_v, rows_v, sem):
        wid = lax.axis_index("s") * NC + lax.axis_index("c")
        base = wid * CH
        pltpu.sync_copy(pos_hbm.at[pl.ds(base, CH)], idx_v)
        pltpu.sync_copy(x_hbm.at[pl.ds(base, CH)], rows_v.at[:, pl.ds(0, K)])
        pltpu.sync_copy(w_hbm.at[pl.ds(base, CH)],
                        rows_v.at[:, pl.ds(K, 128)])
        pltpu.async_copy(rows_v, xs_hbm.at[idx_v], sem).wait()

    @functools.partial(
        pl.kernel,
        out_type=jax.ShapeDtypeStruct((T, N), jnp.float32),
        mesh=mesh,
        scratch_types=[
            pltpu.VMEM((CH,), jnp.int32),
            pltpu.VMEM((CH, N), jnp.float32),
            pltpu.SemaphoreType.DMA,
        ],
    )
    def combine(ys_hbm, pos_hbm, out_hbm, idx_v, rows_v, sem):
        wid = lax.axis_index("s") * NC + lax.axis_index("c")
        base = wid * CH
        pltpu.sync_copy(pos_hbm.at[pl.ds(base, CH)], idx_v)
        pltpu.async_copy(ys_hbm.at[idx_v], rows_v, sem).wait()
        pltpu.sync_copy(rows_v, out_hbm.at[pl.ds(base, CH)])

    xs = dispatch(x, wtok, pos)

    S = 2
    KS_ = K // S
    W1r = W1.reshape(E, S, KS_, N)

    def _w_spec(j):
        return pl.BlockSpec(
            (1, 1, KS_, N), lambda p, gb, ge, fr, j=j: (ge[p], j, 0, 0)
        )

    grid_spec = pltpu.PrefetchScalarGridSpec(
        num_scalar_prefetch=3,
        grid=(P,),
        in_specs=[
            pl.BlockSpec((BLK_M, Kp), lambda p, gb, ge, fr: (gb[p], 0)),
            *[_w_spec(j) for j in range(S)],
            pl.BlockSpec((1, BLK_M, 1), lambda p, gb, ge, fr: (p, 0, 0)),
        ],
        out_specs=pl.BlockSpec((BLK_M, N), lambda p, gb, ge, fr: (gb[p], 0)),
    )
    ys = pl.pallas_call(
        _mm_body,
        grid_spec=grid_spec,
        out_shape=jax.ShapeDtypeStruct((T, N), jnp.float32),
        compiler_params=pltpu.CompilerParams(
            dimension_semantics=("arbitrary",),
        ),
    )(gb, ge, first, xs, *([W1r] * S), seg.reshape(P, BLK_M, 1))

    return combine(ys, pos)
